# Initial kernel scaffold; baseline (speedup 1.0000x reference)
import jax
import jax.numpy as jnp
from jax.experimental import pallas as pl


def kernel(user_table, item_table, edge_vals, edge_index, user_idx, item_idx):
    raise NotImplementedError("write your pallas kernel here")


# speedup vs baseline: 1.0009x; 1.0009x over previous
import jax
import jax.numpy as jnp
from jax.experimental import pallas as pl

NUM_USERS = 50000
NUM_ITEMS = 50000
DIM = 32
N_LAYERS = 2


def _final_body(user_ref, item_ref, rating_ref):
    u = user_ref[...]
    v = item_ref[...]
    rating_ref[...] = jnp.sum(u * v, axis=1)


def kernel(user_table, item_table, edge_vals, edge_index, user_idx, item_idx):
    N = NUM_USERS + NUM_ITEMS
    dst = edge_index[0]
    src = edge_index[1]
    all_emb = jnp.concatenate([user_table, item_table], axis=0)
    acc = all_emb
    emb = all_emb
    for _ in range(N_LAYERS):
        msgs = edge_vals[:, None] * jnp.take(emb, src, axis=0)
        emb = jax.ops.segment_sum(msgs, dst, num_segments=N)
        acc = acc + emb
    light_out = acc / (N_LAYERS + 1)
    user = jnp.take(light_out[:NUM_USERS], user_idx, axis=0)
    item = jnp.take(light_out[NUM_USERS:], item_idx, axis=0)
    rating = pl.pallas_call(
        _final_body,
        out_shape=jax.ShapeDtypeStruct((user.shape[0],), jnp.float32),
    )(user, item)
    return (rating, user, item)


# speedup vs baseline: 7.7715x; 7.7644x over previous
import functools

import jax
import jax.numpy as jnp
from jax import lax
from jax.experimental import pallas as pl
from jax.experimental.pallas import tpu as pltpu
from jax.experimental.pallas import tpu_sc as plsc

NUM_USERS = 50000
NUM_ITEMS = 50000
DIM = 32
N_LAYERS = 2
E = 1600000
B = 4096

NC = 2
NS = 16
L = 16

HALF = 50176
NPAD = 2 * HALF
TRASH = 50000
PADR = HALF - NUM_USERS

K = 512
CHUNKS = 196
EPT = K * CHUNKS
E_PAD = EPT * NS

RPT = HALF // NS
ZB = 112

_mesh = plsc.VectorSubcoreMesh(
    core_axis_name="c", subcore_axis_name="s", num_cores=NC, num_subcores=NS)


def _layer_body(table, srcp, dstp, vals, out,
                acc, zbuf, srcb, dstb, valb, idxb, rowb, gsem):
    c = lax.axis_index("c")
    s = lax.axis_index("s")
    lo = c * HALF

    def _zero(i, _):
        zbuf[i, pl.ds(0, L)] = jnp.zeros((L,), jnp.float32)
        zbuf[i, pl.ds(L, L)] = jnp.zeros((L,), jnp.float32)
        return 0
    lax.fori_loop(0, ZB, _zero, 0)

    def _zcp(i, _):
        pltpu.sync_copy(zbuf, acc.at[pl.ds(s * RPT + i * ZB, ZB)])
        return 0
    lax.fori_loop(0, RPT // ZB, _zcp, 0)
    plsc.subcore_barrier()

    ebase = s * EPT

    def _chunk(g, _):
        base = ebase + g * K
        pltpu.sync_copy(srcp.at[pl.ds(base, K)], srcb)
        pltpu.sync_copy(dstp.at[pl.ds(base, K)], dstb)
        pltpu.sync_copy(vals.at[pl.ds(base, K)], valb)
        for grp in range(K // L):
            d = dstb[pl.ds(grp * L, L)]
            m = (d >= lo) & (d < lo + HALF)
            li = jnp.where(m, d - lo, TRASH)
            idxb[grp * L // 128, pl.ds((grp * L) % 128, L)] = li
        descs = [
            pltpu.async_copy(table.at[srcb.at[pl.ds(b * 128, 128)]],
                             rowb.at[pl.ds(b * 128, 128)], gsem)
            for b in range(4)
        ]
        for dsc in descs:
            dsc.wait()

        def _scale(g2, _):
            wv = valb[pl.ds(g2 * L, L)]
            for j in range(L):
                e = g2 * L + j
                w = wv[j]
                rowb[e, pl.ds(0, L)] = rowb[e, pl.ds(0, L)] * w
                rowb[e, pl.ds(L, L)] = rowb[e, pl.ds(L, L)] * w
            return 0
        lax.fori_loop(0, K // L, _scale, 0)

        for b in range(4):
            pltpu.sync_copy(rowb.at[pl.ds(b * 128, 128)],
                            acc.at[idxb.at[b]], add=True)
        return 0

    lax.fori_loop(0, CHUNKS, _chunk, 0)
    plsc.subcore_barrier()

    gbase = c * HALF + s * RPT
    pltpu.sync_copy(acc.at[pl.ds(s * RPT, RPT)], out.at[pl.ds(gbase, RPT)])


_sc_params = pltpu.CompilerParams(use_tc_tiling_on_sc=False)

_layer = functools.partial(
    pl.kernel,
    out_type=jax.ShapeDtypeStruct((NPAD, DIM), jnp.float32),
    mesh=_mesh,
    compiler_params=_sc_params,
    scratch_types=[
        pltpu.VMEM_SHARED((HALF, DIM), jnp.float32),
        pltpu.VMEM((ZB, DIM), jnp.float32),
        pltpu.VMEM((K,), jnp.int32),
        pltpu.VMEM((K,), jnp.int32),
        pltpu.VMEM((K,), jnp.float32),
        pltpu.VMEM((4, 128), jnp.int32),
        pltpu.VMEM((K, DIM), jnp.float32),
        pltpu.SemaphoreType.DMA,
    ],
)(_layer_body)

BPT = B // (NC * NS)


def _final_body(e0, e1, e2, uidx, iidx, user_out, item_out,
                idxv, r0, r1, r2, ob, gsem):
    c = lax.axis_index("c")
    s = lax.axis_index("s")
    wid = s * NC + c
    third = jnp.float32(1.0 / 3.0)

    for which, idx_hbm, out_hbm in ((0, uidx, user_out), (1, iidx, item_out)):
        pltpu.sync_copy(idx_hbm.at[pl.ds(wid * BPT, BPT)], idxv)
        descs = [pltpu.async_copy(t.at[idxv], r, gsem)
                 for t, r in ((e0, r0), (e1, r1), (e2, r2))]
        for dsc in descs:
            dsc.wait()

        def _avg(g2, _):
            for j in range(4):
                e = g2 * 4 + j
                for h in (0, L):
                    v = (r0[e, pl.ds(h, L)] + r1[e, pl.ds(h, L)]
                         + r2[e, pl.ds(h, L)]) * third
                    ob[e, pl.ds(h, L)] = v
            return 0
        lax.fori_loop(0, BPT // 4, _avg, 0)
        pltpu.sync_copy(ob, out_hbm.at[pl.ds(wid * BPT, BPT)])


_final = functools.partial(
    pl.kernel,
    out_type=(jax.ShapeDtypeStruct((B, DIM), jnp.float32),
              jax.ShapeDtypeStruct((B, DIM), jnp.float32)),
    mesh=_mesh,
    compiler_params=_sc_params,
    scratch_types=[
        pltpu.VMEM((BPT,), jnp.int32),
        pltpu.VMEM((BPT, DIM), jnp.float32),
        pltpu.VMEM((BPT, DIM), jnp.float32),
        pltpu.VMEM((BPT, DIM), jnp.float32),
        pltpu.VMEM((BPT, DIM), jnp.float32),
        pltpu.SemaphoreType.DMA,
    ],
)(_final_body)


def _rating_body(user_ref, item_ref, rating_ref):
    rating_ref[...] = jnp.sum(user_ref[...] * item_ref[...], axis=1)


def kernel(user_table, item_table, edge_vals, edge_index, user_idx, item_idx):
    dst = edge_index[0].astype(jnp.int32)
    src = edge_index[1].astype(jnp.int32)
    srcp = src + PADR * (src >= NUM_USERS).astype(jnp.int32)
    dstp = dst + PADR * (dst >= NUM_USERS).astype(jnp.int32)
    pad = E_PAD - E
    srcp = jnp.concatenate([srcp, jnp.zeros((pad,), jnp.int32)])
    dstp = jnp.concatenate([dstp, jnp.zeros((pad,), jnp.int32)])
    vals = jnp.concatenate([edge_vals, jnp.zeros((pad,), jnp.float32)])
    zp = jnp.zeros((PADR, DIM), jnp.float32)
    e0 = jnp.concatenate([user_table, zp, item_table, zp], axis=0)

    e1 = _layer(e0, srcp, dstp, vals)
    e2 = _layer(e1, srcp, dstp, vals)

    uidx = user_idx.astype(jnp.int32)
    iidx = item_idx.astype(jnp.int32) + HALF
    user, item = _final(e0, e1, e2, uidx, iidx)
    rating = pl.pallas_call(
        _rating_body,
        out_shape=jax.ShapeDtypeStruct((B,), jnp.float32),
    )(user, item)
    return (rating, user, item)


# speedup vs baseline: 7.7994x; 1.0036x over previous
import functools

import jax
import jax.numpy as jnp
from jax import lax
from jax.experimental import pallas as pl
from jax.experimental.pallas import tpu as pltpu
from jax.experimental.pallas import tpu_sc as plsc

NUM_USERS = 50000
NUM_ITEMS = 50000
DIM = 32
N_LAYERS = 2
E = 1600000
B = 4096

NC = 2
NS = 16
L = 16

HALF = 50176
NPAD = 2 * HALF
TRASH = 50000
PADR = HALF - NUM_USERS

K = 512
CHUNKS = 196
EPT = K * CHUNKS
E_PAD = EPT * NS

RPT = HALF // NS
ZB = 112

_mesh = plsc.VectorSubcoreMesh(
    core_axis_name="c", subcore_axis_name="s", num_cores=NC, num_subcores=NS)


def _layer_body(table, srcp, dstp, vals, out,
                acc, zbuf, srcb0, dstb0, valb0, srcb1, dstb1, valb1,
                csrcf, cvalf, cidxf, cidx, rowb,
                psem, gs0, gs1, gs2, gs3):
    c = lax.axis_index("c")
    s = lax.axis_index("s")
    lo = c * HALF
    gsems = (gs0, gs1, gs2, gs3)

    def _zero(i, _):
        zbuf[i, pl.ds(0, L)] = jnp.zeros((L,), jnp.float32)
        zbuf[i, pl.ds(L, L)] = jnp.zeros((L,), jnp.float32)
        return 0
    lax.fori_loop(0, ZB, _zero, 0)

    def _zcp(i, _):
        pltpu.sync_copy(zbuf, acc.at[pl.ds(s * RPT + i * ZB, ZB)])
        return 0
    lax.fori_loop(0, RPT // ZB, _zcp, 0)
    plsc.subcore_barrier()

    ebase = s * EPT
    bufs = ((srcb0, dstb0, valb0), (srcb1, dstb1, valb1))

    def _prefetch(g, par):
        base = ebase + g * K
        sb, db, vb = bufs[par]
        pltpu.async_copy(srcp.at[pl.ds(base, K)], sb, psem)
        pltpu.async_copy(dstp.at[pl.ds(base, K)], db, psem)
        pltpu.async_copy(vals.at[pl.ds(base, K)], vb, psem)

    def _drain(par):
        sb, db, vb = bufs[par]
        pltpu.make_async_copy(srcp.at[pl.ds(0, K)], sb, psem).wait()
        pltpu.make_async_copy(dstp.at[pl.ds(0, K)], db, psem).wait()
        pltpu.make_async_copy(vals.at[pl.ds(0, K)], vb, psem).wait()

    _prefetch(0, 0)

    def _chunk2(g2, _):
        for par in range(2):
            g = g2 * 2 + par
            sb, db, vb = bufs[par]
            _drain(par)
            _prefetch(g + 1, 1 - par)

            for grp in range(K // L):
                d = db[pl.ds(grp * L, L)]
                m = (d >= lo) & (d < lo + HALF)
                li = jnp.where(m, d - lo, TRASH)
                cidxf[pl.ds(grp * L, L)] = li
                csrcf[pl.ds(grp * L, L)] = sb[pl.ds(grp * L, L)]
                cvalf[pl.ds(grp * L, L)] = vb[pl.ds(grp * L, L)]
                cidx[grp * L // 128, pl.ds((grp * L) % 128, L)] = li

            descs = [
                pltpu.async_copy(table.at[csrcf.at[pl.ds(b * 128, 128)]],
                                 rowb.at[pl.ds(b * 128, 128)], gsems[b])
                for b in range(4)
            ]
            for b in range(4):
                descs[b].wait()

                def _scale(g3, _):
                    e0_ = b * 128 + g3 * L
                    wv = cvalf[pl.ds(e0_, L)]
                    for j in range(L):
                        e = e0_ + j
                        w = wv[j]
                        rowb[e, pl.ds(0, L)] = rowb[e, pl.ds(0, L)] * w
                        rowb[e, pl.ds(L, L)] = rowb[e, pl.ds(L, L)] * w
                    return 0
                lax.fori_loop(0, 128 // L, _scale, 0)
                pltpu.sync_copy(rowb.at[pl.ds(b * 128, 128)],
                                acc.at[cidx.at[b]], add=True)
        return 0

    lax.fori_loop(0, CHUNKS // 2, _chunk2, 0)
    _drain(0)
    plsc.subcore_barrier()

    gbase = c * HALF + s * RPT
    pltpu.sync_copy(acc.at[pl.ds(s * RPT, RPT)], out.at[pl.ds(gbase, RPT)])


_sc_params = pltpu.CompilerParams(
    use_tc_tiling_on_sc=False, needs_layout_passes=False)

_layer = functools.partial(
    pl.kernel,
    out_type=jax.ShapeDtypeStruct((NPAD, DIM), jnp.float32),
    mesh=_mesh,
    compiler_params=_sc_params,
    scratch_types=[
        pltpu.VMEM_SHARED((HALF, DIM), jnp.float32),
        pltpu.VMEM((ZB, DIM), jnp.float32),
        pltpu.VMEM((K,), jnp.int32),
        pltpu.VMEM((K,), jnp.int32),
        pltpu.VMEM((K,), jnp.float32),
        pltpu.VMEM((K,), jnp.int32),
        pltpu.VMEM((K,), jnp.int32),
        pltpu.VMEM((K,), jnp.float32),
        pltpu.VMEM((K + L,), jnp.int32),
        pltpu.VMEM((K + L,), jnp.float32),
        pltpu.VMEM((K + L,), jnp.int32),
        pltpu.VMEM((4, 128), jnp.int32),
        pltpu.VMEM((K, DIM), jnp.float32),
        pltpu.SemaphoreType.DMA,
        pltpu.SemaphoreType.DMA,
        pltpu.SemaphoreType.DMA,
        pltpu.SemaphoreType.DMA,
        pltpu.SemaphoreType.DMA,
    ],
)(_layer_body)

BPT = B // (NC * NS)


def _final_body(e0, e1, e2, uidx, iidx, user_out, item_out,
                idxv, r0, r1, r2, ob, gsem):
    c = lax.axis_index("c")
    s = lax.axis_index("s")
    wid = s * NC + c
    third = jnp.float32(1.0 / 3.0)

    for which, idx_hbm, out_hbm in ((0, uidx, user_out), (1, iidx, item_out)):
        pltpu.sync_copy(idx_hbm.at[pl.ds(wid * BPT, BPT)], idxv)
        descs = [pltpu.async_copy(t.at[idxv], r, gsem)
                 for t, r in ((e0, r0), (e1, r1), (e2, r2))]
        for dsc in descs:
            dsc.wait()

        def _avg(g2, _):
            for j in range(4):
                e = g2 * 4 + j
                for h in (0, L):
                    v = (r0[e, pl.ds(h, L)] + r1[e, pl.ds(h, L)]
                         + r2[e, pl.ds(h, L)]) * third
                    ob[e, pl.ds(h, L)] = v
            return 0
        lax.fori_loop(0, BPT // 4, _avg, 0)
        pltpu.sync_copy(ob, out_hbm.at[pl.ds(wid * BPT, BPT)])


_final = functools.partial(
    pl.kernel,
    out_type=(jax.ShapeDtypeStruct((B, DIM), jnp.float32),
              jax.ShapeDtypeStruct((B, DIM), jnp.float32)),
    mesh=_mesh,
    compiler_params=_sc_params,
    scratch_types=[
        pltpu.VMEM((BPT,), jnp.int32),
        pltpu.VMEM((BPT, DIM), jnp.float32),
        pltpu.VMEM((BPT, DIM), jnp.float32),
        pltpu.VMEM((BPT, DIM), jnp.float32),
        pltpu.VMEM((BPT, DIM), jnp.float32),
        pltpu.SemaphoreType.DMA,
    ],
)(_final_body)


def _rating_body(user_ref, item_ref, rating_ref):
    rating_ref[...] = jnp.sum(user_ref[...] * item_ref[...], axis=1)


def kernel(user_table, item_table, edge_vals, edge_index, user_idx, item_idx):
    dst = edge_index[0].astype(jnp.int32)
    src = edge_index[1].astype(jnp.int32)
    srcp = src + PADR * (src >= NUM_USERS).astype(jnp.int32)
    dstp = dst + PADR * (dst >= NUM_USERS).astype(jnp.int32)
    pad = E_PAD - E + K
    srcp = jnp.concatenate([srcp, jnp.zeros((pad,), jnp.int32)])
    dstp = jnp.concatenate([dstp, jnp.zeros((pad,), jnp.int32)])
    vals = jnp.concatenate([edge_vals, jnp.zeros((pad,), jnp.float32)])
    zp = jnp.zeros((PADR, DIM), jnp.float32)
    e0 = jnp.concatenate([user_table, zp, item_table, zp], axis=0)

    e1 = _layer(e0, srcp, dstp, vals)
    e2 = _layer(e1, srcp, dstp, vals)

    uidx = user_idx.astype(jnp.int32)
    iidx = item_idx.astype(jnp.int32) + HALF
    user, item = _final(e0, e1, e2, uidx, iidx)
    rating = pl.pallas_call(
        _rating_body,
        out_shape=jax.ShapeDtypeStruct((B,), jnp.float32),
    )(user, item)
    return (rating, user, item)
